# baseline (speedup 1.0000x reference)
import jax
import jax.numpy as jnp
from jax.experimental import pallas as pl
from jax.experimental.pallas import tpu as pltpu
from jax.experimental.pallas import tpu_sc as plsc

_N_LAT, _N_LON, _N_MESH, _N_NEI = 64, 128, 128, 4
_IN_DIM, _HID, _OUT_CH = 256, 256, 78
_BATCH = 32
_N_GRID = _N_LAT * _N_LON
_MBB = 8
_N_WORKERS = 32
_ROWS_PER_W = _N_GRID // _N_WORKERS


def _sc_build_a(idx_hbm, wts_hbm, z_hbm, a_hbm, ibuf, wbuf, abuf):
    c = jax.lax.axis_index("c")
    s = jax.lax.axis_index("s")
    wid = s * 2 + c
    base = wid * _ROWS_PER_W
    pltpu.sync_copy(z_hbm, abuf)
    pltpu.sync_copy(idx_hbm.at[:, pl.ds(base, _ROWS_PER_W)], ibuf)
    pltpu.sync_copy(wts_hbm.at[:, pl.ds(base, _ROWS_PER_W)], wbuf)
    lbase = jax.lax.iota(jnp.int32, 16) * _N_MESH
    for k in range(_N_NEI):
        for j in range(_ROWS_PER_W // 16):
            cols = ibuf[k, pl.ds(j * 16, 16)]
            vals = wbuf[k, pl.ds(j * 16, 16)]
            addr = j * (16 * _N_MESH) + lbase + cols
            plsc.store_scatter(abuf, [addr], vals)
    pltpu.sync_copy(abuf, a_hbm.at[wid])


def _mlp_body(nf_ref, w1_ref, b1_ref, w2_ref, b2_ref, out_ref):
    x = nf_ref[...].reshape(_MBB * _N_MESH, _IN_DIM)
    h = jnp.dot(x, w1_ref[...], preferred_element_type=jnp.float32)
    h = jnp.maximum(h + b1_ref[...], 0.0)
    o = jnp.dot(h, w2_ref[...], preferred_element_type=jnp.float32)
    o = o + b2_ref[...]
    out_ref[...] = o.reshape(_MBB, _N_MESH, _OUT_CH).astype(jnp.bfloat16)


def _interp_body(a_ref, mesh_ref, out_ref, a_s):
    s = pl.program_id(0)

    @pl.when(s == 0)
    def _cast():
        a_s[...] = a_ref[...].astype(jnp.bfloat16)

    r = jax.lax.dot_general(
        a_s[...], mesh_ref[0],
        (((1,), (0,)), ((), ())),
        preferred_element_type=jnp.float32)
    out_ref[...] = r.reshape(1, _N_LAT, _N_LON, _OUT_CH)


def kernel(node_features, W1, b1, W2, b2, neighbor_indices, neighbor_weights):
    idx_t = neighbor_indices.T
    wts_t = neighbor_weights.T
    zblock = jnp.zeros((_ROWS_PER_W * _N_MESH,), jnp.float32)

    build = pl.kernel(
        _sc_build_a,
        out_type=jax.ShapeDtypeStruct((_N_WORKERS, _ROWS_PER_W * _N_MESH),
                                      jnp.float32),
        mesh=plsc.VectorSubcoreMesh(core_axis_name="c", subcore_axis_name="s"),
        scratch_types=[
            pltpu.VMEM((_N_NEI, _ROWS_PER_W), jnp.int32),
            pltpu.VMEM((_N_NEI, _ROWS_PER_W), jnp.float32),
            pltpu.VMEM((_ROWS_PER_W * _N_MESH,), jnp.float32),
        ],
        compiler_params=pltpu.CompilerParams(needs_layout_passes=False),
    )
    a_mat = build(idx_t, wts_t, zblock).reshape(_N_GRID, _N_MESH)

    mesh = pl.pallas_call(
        _mlp_body,
        grid=(_BATCH // _MBB,),
        in_specs=[
            pl.BlockSpec((_MBB, _N_MESH, _IN_DIM), lambda i: (i, 0, 0)),
            pl.BlockSpec((_IN_DIM, _HID), lambda i: (0, 0)),
            pl.BlockSpec((1, _HID), lambda i: (0, 0)),
            pl.BlockSpec((_HID, _OUT_CH), lambda i: (0, 0)),
            pl.BlockSpec((1, _OUT_CH), lambda i: (0, 0)),
        ],
        out_specs=pl.BlockSpec((_MBB, _N_MESH, _OUT_CH), lambda i: (i, 0, 0)),
        out_shape=jax.ShapeDtypeStruct((_BATCH, _N_MESH, _OUT_CH),
                                       jnp.bfloat16),
        compiler_params=pltpu.CompilerParams(
            dimension_semantics=("parallel",)),
    )(node_features, W1, b1.reshape(1, _HID), W2, b2.reshape(1, _OUT_CH))

    out = pl.pallas_call(
        _interp_body,
        grid=(_BATCH,),
        in_specs=[
            pl.BlockSpec((_N_GRID, _N_MESH), lambda b: (0, 0)),
            pl.BlockSpec((1, _N_MESH, _OUT_CH), lambda b: (b, 0, 0)),
        ],
        out_specs=pl.BlockSpec((1, _N_LAT, _N_LON, _OUT_CH),
                               lambda b: (b, 0, 0, 0)),
        out_shape=jax.ShapeDtypeStruct((_BATCH, _N_LAT, _N_LON, _OUT_CH),
                                       jnp.float32),
        scratch_shapes=[pltpu.VMEM((_N_GRID, _N_MESH), jnp.bfloat16)],
        compiler_params=pltpu.CompilerParams(
            dimension_semantics=("arbitrary",)),
    )(a_mat, mesh)

    return out


# speedup vs baseline: 1.1317x; 1.1317x over previous
import jax
import jax.numpy as jnp
from jax.experimental import pallas as pl
from jax.experimental.pallas import tpu as pltpu

_N_LAT, _N_LON, _N_MESH, _N_NEI = 64, 128, 128, 4
_IN_DIM, _HID, _OUT_CH = 256, 256, 78
_BATCH = 32
_N_GRID = _N_LAT * _N_LON


def _fused_body(nf_ref, w1_ref, b1_ref, w2_ref, b2_ref, idx_ref, wts_ref,
                out_ref, mesh_s, a_s):
    s = pl.program_id(0)

    @pl.when(s == 0)
    def _prep():
        for c in range(4):
            x = nf_ref[c * 8:(c + 1) * 8].reshape(8 * _N_MESH, _IN_DIM)
            h = jnp.dot(x, w1_ref[...],
                        preferred_element_type=jnp.float32)
            h = jnp.maximum(h + b1_ref[...], 0.0)
            o = jnp.dot(h, w2_ref[...],
                        preferred_element_type=jnp.float32)
            o = o + b2_ref[...]
            mesh_s[c * 8:(c + 1) * 8] = (
                o.reshape(8, _N_MESH, _OUT_CH).astype(jnp.bfloat16))

        iota = jax.lax.broadcasted_iota(
            jnp.int32, (_N_GRID // 4, _N_MESH), 1).astype(jnp.bfloat16)
        zero = jnp.zeros((), jnp.bfloat16)
        for c in range(4):
            rows = pl.ds(c * (_N_GRID // 4), _N_GRID // 4)
            acc = jnp.zeros((_N_GRID // 4, _N_MESH), jnp.bfloat16)
            for k in range(_N_NEI):
                idxb = idx_ref[rows, k:k + 1].astype(jnp.bfloat16)
                wb = wts_ref[rows, k:k + 1].astype(jnp.bfloat16)
                acc = acc + jnp.where(idxb == iota, wb, zero)
            a_s[rows] = acc

    @pl.when(s > 0)
    def _interp():
        b = s - 1
        r = jax.lax.dot_general(
            a_s[...], mesh_s[b],
            (((1,), (0,)), ((), ())),
            preferred_element_type=jnp.float32)
        out_ref[...] = r.reshape(1, _N_LAT, _N_LON, _OUT_CH)


def kernel(node_features, W1, b1, W2, b2, neighbor_indices, neighbor_weights):
    out = pl.pallas_call(
        _fused_body,
        grid=(_BATCH + 1,),
        in_specs=[
            pl.BlockSpec((_BATCH, _N_MESH, _IN_DIM), lambda s: (0, 0, 0)),
            pl.BlockSpec((_IN_DIM, _HID), lambda s: (0, 0)),
            pl.BlockSpec((1, _HID), lambda s: (0, 0)),
            pl.BlockSpec((_HID, _OUT_CH), lambda s: (0, 0)),
            pl.BlockSpec((1, _OUT_CH), lambda s: (0, 0)),
            pl.BlockSpec((_N_GRID, _N_NEI), lambda s: (0, 0)),
            pl.BlockSpec((_N_GRID, _N_NEI), lambda s: (0, 0)),
        ],
        out_specs=pl.BlockSpec((1, _N_LAT, _N_LON, _OUT_CH),
                               lambda s: (jnp.maximum(s - 1, 0), 0, 0, 0)),
        out_shape=jax.ShapeDtypeStruct((_BATCH, _N_LAT, _N_LON, _OUT_CH),
                                       jnp.float32),
        scratch_shapes=[
            pltpu.VMEM((_BATCH, _N_MESH, _OUT_CH), jnp.bfloat16),
            pltpu.VMEM((_N_GRID, _N_MESH), jnp.bfloat16),
        ],
        compiler_params=pltpu.CompilerParams(
            dimension_semantics=("arbitrary",)),
    )(node_features, W1, b1.reshape(1, _HID), W2, b2.reshape(1, _OUT_CH),
      neighbor_indices, neighbor_weights)

    return out


# speedup vs baseline: 1.1562x; 1.0216x over previous
import jax
import jax.numpy as jnp
from jax.experimental import pallas as pl
from jax.experimental.pallas import tpu as pltpu

_N_LAT, _N_LON, _N_MESH, _N_NEI = 64, 128, 128, 4
_IN_DIM, _HID, _OUT_CH = 256, 256, 78
_BATCH = 32
_N_GRID = _N_LAT * _N_LON
_N_STEPS = 8
_CHUNK = _N_GRID // _N_STEPS
_CLAT = _N_LAT // _N_STEPS


def _fused_body(nf_ref, w1_ref, b1_ref, w2_ref, b2_ref, idx_ref, wts_ref,
                out_ref, mesh_s):
    t = pl.program_id(0)

    @pl.when(t == 0)
    def _mlp():
        for c in range(4):
            x = nf_ref[c * 8:(c + 1) * 8].reshape(8 * _N_MESH, _IN_DIM)
            h = jnp.dot(x, w1_ref[...], preferred_element_type=jnp.float32)
            h = jnp.maximum(h + b1_ref[...], 0.0)
            o = jnp.dot(h, w2_ref[...], preferred_element_type=jnp.float32)
            o = o + b2_ref[...]
            mesh_s[c * 8:(c + 1) * 8] = (
                o.reshape(8, _N_MESH, _OUT_CH).astype(jnp.bfloat16))

    iota = jax.lax.broadcasted_iota(
        jnp.int32, (_CHUNK, _N_MESH), 1).astype(jnp.bfloat16)
    zero = jnp.zeros((), jnp.bfloat16)
    a_c = jnp.zeros((_CHUNK, _N_MESH), jnp.bfloat16)
    for k in range(_N_NEI):
        idxb = idx_ref[:, k:k + 1].astype(jnp.bfloat16)
        wb = wts_ref[:, k:k + 1].astype(jnp.bfloat16)
        a_c = a_c + jnp.where(idxb == iota, wb, zero)

    for b in range(_BATCH):
        r = jax.lax.dot_general(
            a_c, mesh_s[b],
            (((1,), (0,)), ((), ())),
            preferred_element_type=jnp.float32)
        out_ref[b] = r.reshape(_CLAT, _N_LON, _OUT_CH)


def kernel(node_features, W1, b1, W2, b2, neighbor_indices, neighbor_weights):
    out = pl.pallas_call(
        _fused_body,
        grid=(_N_STEPS,),
        in_specs=[
            pl.BlockSpec((_BATCH, _N_MESH, _IN_DIM), lambda t: (0, 0, 0)),
            pl.BlockSpec((_IN_DIM, _HID), lambda t: (0, 0)),
            pl.BlockSpec((1, _HID), lambda t: (0, 0)),
            pl.BlockSpec((_HID, _OUT_CH), lambda t: (0, 0)),
            pl.BlockSpec((1, _OUT_CH), lambda t: (0, 0)),
            pl.BlockSpec((_CHUNK, _N_NEI), lambda t: (t, 0)),
            pl.BlockSpec((_CHUNK, _N_NEI), lambda t: (t, 0)),
        ],
        out_specs=pl.BlockSpec((_BATCH, _CLAT, _N_LON, _OUT_CH),
                               lambda t: (0, t, 0, 0)),
        out_shape=jax.ShapeDtypeStruct((_BATCH, _N_LAT, _N_LON, _OUT_CH),
                                       jnp.float32),
        scratch_shapes=[
            pltpu.VMEM((_BATCH, _N_MESH, _OUT_CH), jnp.bfloat16),
        ],
        compiler_params=pltpu.CompilerParams(
            dimension_semantics=("arbitrary",)),
    )(node_features, W1, b1.reshape(1, _HID), W2, b2.reshape(1, _OUT_CH),
      neighbor_indices, neighbor_weights)

    return out
